# baseline (speedup 1.0000x reference)
import functools

import numpy as np

import jax
import jax.numpy as jnp
from jax import lax
from jax.experimental import pallas as pl
from jax.experimental.pallas import tpu as pltpu
from jax.experimental.pallas import tpu_sc as plsc

_N = 10000
_E = 320000
_D = 128
_H = 128
_ET = 4

_NC = 2
_NS = 16
_L = 16
_NW = _NC * _NS
_EPW = _E // _NW
_K = 40
_NCH = _EPW // _K
_SUP = 10
_NSUP = _NCH // _SUP
_NP = 10240
_RPT = _NP // _NS
_BLK = 2048
_EB = 8000

_QPERM = np.empty(_H, np.int32)
for _g in range(_H // 32):
    for _i in range(16):
        _QPERM[32 * _g + 2 * _i] = 32 * _g + _i
        _QPERM[32 * _g + 2 * _i + 1] = 32 * _g + 16 + _i


def _sc_edge_pass(T, C, sd_flat, src):
    mesh = plsc.VectorSubcoreMesh(core_axis_name="c", subcore_axis_name="s")

    @functools.partial(
        pl.kernel,
        mesh=mesh,
        out_type=jax.ShapeDtypeStruct((_NC, _NP, _H), jnp.float32),
        scratch_types=[
            pltpu.VMEM((_SUP * 2 * _K,), jnp.int32),
            pltpu.VMEM((_K,), jnp.int32),
            pltpu.VMEM((_K,), jnp.int32),
            pltpu.VMEM((2, 2 * _K, _H), jnp.float32),
            pltpu.VMEM((2, _K, _H // 2), jnp.int32),
            pltpu.VMEM((2, _K, _H), jnp.float32),
            pltpu.VMEM_SHARED((_NP, _H), jnp.float32),
            pltpu.SemaphoreType.DMA,
            pltpu.SemaphoreType.DMA,
            pltpu.SemaphoreType.DMA,
            pltpu.SemaphoreType.DMA,
            pltpu.SemaphoreType.DMA,
            pltpu.SemaphoreType.DMA,
        ],
    )
    def k(t_h, c_h, sd_h, src_h, out_h,
          sdsup, ss0, ss1, bufab, bufc, bufm, acc,
          semg0, semg1, sems0, sems1, semss0, semss1):
        semg = [semg0, semg1]
        sems = [sems0, sems1]
        semss = [semss0, semss1]
        sidxs = [ss0, ss1]
        cid = lax.axis_index("c")
        sid = lax.axis_index("s")
        wid = sid * _NC + cid

        def zrow(r, carry):
            for j in range(_H // _L):
                bufm[0, r, pl.ds(j * _L, _L)] = jnp.zeros((_L,), jnp.float32)
            return carry

        lax.fori_loop(0, _K, zrow, 0)
        for i in range(_RPT // _K):
            pltpu.sync_copy(bufm.at[0], acc.at[pl.ds(sid * _RPT + i * _K, _K)])
        plsc.subcore_barrier()

        sdw = _SUP * 2 * _K
        ebase = wid * _EPW

        def load_super(g):
            pltpu.sync_copy(sd_h.at[pl.ds((wid * _NCH + g * _SUP) * 2 * _K,
                                          sdw)], sdsup)

        def gather_issue(c, r, slot):
            pltpu.async_copy(t_h.at[sdsup.at[pl.ds(r * 2 * _K, 2 * _K)]],
                             bufab.at[slot], semg[slot])
            pltpu.async_copy(c_h.at[pl.ds(ebase + c * _K, _K)],
                             bufc.at[slot], semg[slot])

        def gather_wait(c, r, slot):
            pltpu.make_async_copy(t_h.at[sdsup.at[pl.ds(r * 2 * _K, 2 * _K)]],
                                  bufab.at[slot], semg[slot]).wait()
            pltpu.make_async_copy(c_h.at[pl.ds(ebase + c * _K, _K)],
                                  bufc.at[slot], semg[slot]).wait()

        def step(p, r):
            s = r % 2
            c = p * _SUP + r

            if r < _SUP - 1:
                gather_issue(c + 1, r + 1, 1 - s)

            @pl.when(c >= 2)
            def _():
                pltpu.make_async_copy(
                    bufm.at[s], acc.at[sidxs[s]], sems[s]).wait()

            pltpu.async_copy(src_h.at[pl.ds(ebase + c * _K, _K)],
                             sidxs[s], semss[s])

            gather_wait(c, r, s)

            ba = bufab.at[s]
            bc = bufc.at[s]
            bm = bufm.at[s]

            mhi = jnp.int32(-65536)

            @plsc.parallel_loop(0, _K, unroll=4)
            def row(e):
                e2 = _K + e
                for g in range(_H // (2 * _L)):
                    cw = bc[e, pl.ds(g * _L, _L)]
                    bcf = lambda w: jax.lax.bitcast_convert_type(
                        w, jnp.float32)
                    sllo = pl.ds(g * 2 * _L, _L)
                    slhi = pl.ds(g * 2 * _L + _L, _L)
                    lo = ba[e, sllo] + ba[e2, sllo] + bcf(cw << 16)
                    hi = ba[e, slhi] + ba[e2, slhi] + bcf(cw & mhi)
                    bm[e, sllo] = jnp.maximum(lo, 0.0)
                    bm[e, slhi] = jnp.maximum(hi, 0.0)

            pltpu.make_async_copy(src_h.at[pl.ds(ebase + c * _K, _K)],
                                  sidxs[s], semss[s]).wait()
            pltpu.async_copy(bufm.at[s], acc.at[sidxs[s]], sems[s], add=True)

            if r == _SUP - 1:
                @pl.when(p + 1 < _NSUP)
                def _():
                    load_super(p + 1)
                    gather_issue(c + 1, 0, 1 - s)

        load_super(0)
        gather_issue(0, 0, 0)

        def superchunk(p, carry):
            for r in range(_SUP):
                step(p, r)
            return carry

        lax.fori_loop(0, _NSUP, superchunk, 0)
        pltpu.make_async_copy(bufm.at[0], acc.at[sidxs[0]], sems[0]).wait()
        pltpu.make_async_copy(bufm.at[1], acc.at[sidxs[1]], sems[1]).wait()
        plsc.subcore_barrier()
        pltpu.sync_copy(acc.at[pl.ds(sid * _RPT, _RPT)],
                        out_h.at[cid, pl.ds(sid * _RPT, _RPT)])

    return k(T, C, sd_flat, src)


def _tc_edge(ef, wc0, b0, wc1, b1):
    def body(ef_ref, w0_ref, b0_ref, w1_ref, b1_ref, c0_ref, c1_ref):
        e = ef_ref[...]
        c0_ref[...] = (jnp.dot(e, w0_ref[...],
                               preferred_element_type=jnp.float32)
                       + b0_ref[...]).astype(jnp.bfloat16)
        c1_ref[...] = (jnp.dot(e, w1_ref[...],
                               preferred_element_type=jnp.float32)
                       + b1_ref[...]).astype(jnp.bfloat16)

    wfull = pl.BlockSpec((_ET, _H), lambda i: (0, 0))
    row1 = pl.BlockSpec((1, _H), lambda i: (0, 0))
    eblk = pl.BlockSpec((_EB, _ET), lambda i: (i, 0))
    cblk = pl.BlockSpec((_EB, _H), lambda i: (i, 0))
    return pl.pallas_call(
        body,
        grid=(_E // _EB,),
        in_specs=[eblk, wfull, row1, wfull, row1],
        out_specs=[cblk, cblk],
        out_shape=[
            jax.ShapeDtypeStruct((_E, _H), jnp.bfloat16),
            jax.ShapeDtypeStruct((_E, _H), jnp.bfloat16),
        ],
    )(ef, wc0, b0, wc1, b1)


def _tc_prep(x, w_child, b_child, wa, wb):
    def body(x_ref, wc_ref, bc_ref, wa_ref, wb_ref, t_ref, p_ref):
        i = pl.program_id(0)
        cf = jnp.maximum(
            jnp.dot(x_ref[...], wc_ref[...],
                    preferred_element_type=jnp.float32) + bc_ref[...], 0.0)
        t_ref[0] = jnp.dot(cf, wa_ref[...], preferred_element_type=jnp.float32)
        t_ref[1] = jnp.dot(cf, wb_ref[...], preferred_element_type=jnp.float32)
        s = jnp.sum(cf, axis=0, keepdims=True)

        @pl.when(i == 0)
        def _():
            p_ref[...] = s

        @pl.when(i != 0)
        def _():
            p_ref[...] = p_ref[...] + s

    full = pl.BlockSpec((_H, _H), lambda i: (0, 0))
    row1 = pl.BlockSpec((1, _H), lambda i: (0, 0))
    nblk = pl.BlockSpec((_BLK, _H), lambda i: (i, 0))
    tblk = pl.BlockSpec((2, _BLK, _H), lambda i: (0, i, 0))
    return pl.pallas_call(
        body,
        grid=(_NP // _BLK,),
        in_specs=[nblk, full, row1, full, full],
        out_specs=[tblk, row1],
        out_shape=[
            jax.ShapeDtypeStruct((2, _NP, _H), jnp.float32),
            jax.ShapeDtypeStruct((1, _H), jnp.float32),
        ],
    )(x, w_child, b_child, wa, wb)


def _tc_mid(parts, wa, wb):
    def body(p_ref, wa_ref, wb_ref, t_ref, s_ref):
        i = pl.program_id(0)
        cf = p_ref[0] + p_ref[1]
        t_ref[0] = jnp.dot(cf, wa_ref[...], preferred_element_type=jnp.float32)
        t_ref[1] = jnp.dot(cf, wb_ref[...], preferred_element_type=jnp.float32)
        s = jnp.sum(cf, axis=0, keepdims=True)

        @pl.when(i == 0)
        def _():
            s_ref[...] = s

        @pl.when(i != 0)
        def _():
            s_ref[...] = s_ref[...] + s

    full = pl.BlockSpec((_H, _H), lambda i: (0, 0))
    row1 = pl.BlockSpec((1, _H), lambda i: (0, 0))
    pblk = pl.BlockSpec((2, _BLK, _H), lambda i: (0, i, 0))
    tblk = pl.BlockSpec((2, _BLK, _H), lambda i: (0, i, 0))
    return pl.pallas_call(
        body,
        grid=(_NP // _BLK,),
        in_specs=[pblk, full, full],
        out_specs=[tblk, row1],
        out_shape=[
            jax.ShapeDtypeStruct((2, _NP, _H), jnp.float32),
            jax.ShapeDtypeStruct((1, _H), jnp.float32),
        ],
    )(parts, wa, wb)


def _tc_fin(parts, p0, p1, wp0, wp1, wp2, bp):
    def body(parts_ref, p0_ref, p1_ref, w0_ref, w1_ref, w2_ref, bp_ref,
             out_ref, acc_ref):
        i = pl.program_id(0)
        s = jnp.sum(parts_ref[0] + parts_ref[1], axis=0, keepdims=True)

        @pl.when(i == 0)
        def _():
            acc_ref[...] = s

        @pl.when(i != 0)
        def _():
            acc_ref[...] = acc_ref[...] + s

        @pl.when(i == pl.num_programs(0) - 1)
        def _():
            r = jnp.dot(p0_ref[...], w0_ref[...],
                        preferred_element_type=jnp.float32)
            r = r + jnp.dot(p1_ref[...], w1_ref[...],
                            preferred_element_type=jnp.float32)
            r = r + jnp.dot(acc_ref[...], w2_ref[...],
                            preferred_element_type=jnp.float32)
            out_ref[...] = jnp.maximum(r + bp_ref[...], 0.0)

    full = pl.BlockSpec((_H, _H), lambda i: (0, 0))
    row1 = pl.BlockSpec((1, _H), lambda i: (0, 0))
    pblk = pl.BlockSpec((2, _BLK, _H), lambda i: (0, i, 0))
    return pl.pallas_call(
        body,
        grid=(_NP // _BLK,),
        in_specs=[pblk, row1, row1, full, full, full, row1],
        out_specs=row1,
        out_shape=jax.ShapeDtypeStruct((1, _D), jnp.float32),
        scratch_shapes=[pltpu.VMEM((1, _H), jnp.float32)],
    )(parts, p0, p1, wp0, wp1, wp2, bp)


def kernel(child_feats, child_exists, edge_type_onehot, edge_indices,
           W_child, b_child, W_ne0, b_ne0, W_ne1, b_ne1, W_parent, b_parent):
    x = (child_feats * child_exists)[0]
    x = jnp.concatenate([x, jnp.zeros((_NP - _N, _D), jnp.float32)], axis=0)
    src = edge_indices[0, :, 0]
    dst = edge_indices[0, :, 1]
    sd_flat = jnp.concatenate(
        [src.reshape(_NW, _NCH, _K), dst.reshape(_NW, _NCH, _K) + _NP],
        axis=2).reshape(_E * 2)

    wa0, wb0 = W_ne0[:_H], W_ne0[_H:2 * _H]
    wa1, wb1 = W_ne1[:_H], W_ne1[_H:2 * _H]

    c0, c1 = _tc_edge(edge_type_onehot[0], W_ne0[2 * _H:, _QPERM],
                      b_ne0[_QPERM].reshape(1, _H), W_ne1[2 * _H:, _QPERM],
                      b_ne1[_QPERM].reshape(1, _H))
    def _i32view(a, rows):
        return jax.lax.bitcast_convert_type(
            a.reshape(rows, _H // 2, 2), jnp.int32)

    c0v = _i32view(c0, _E)
    c1v = _i32view(c1, _E)
    t0, p0 = _tc_prep(x, W_child, b_child.reshape(1, _H), wa0, wb0)
    parts1 = _sc_edge_pass(t0.reshape(2 * _NP, _H), c0v, sd_flat, src)
    t1, p1 = _tc_mid(parts1, wa1, wb1)
    parts2 = _sc_edge_pass(t1.reshape(2 * _NP, _H), c1v, sd_flat, src)
    return _tc_fin(parts2, p0, p1,
                   W_parent[:_H], W_parent[_H:2 * _H], W_parent[2 * _H:],
                   b_parent.reshape(1, _D))


# speedup vs baseline: 3.3164x; 3.3164x over previous
import functools

import jax
import jax.numpy as jnp
from jax import lax
from jax.experimental import pallas as pl
from jax.experimental.pallas import tpu as pltpu
from jax.experimental.pallas import tpu_sc as plsc

_N = 10000
_E = 320000
_D = 128
_H = 128
_ET = 4

_NC = 2
_NS = 16
_L = 16
_NW = _NC * _NS
_EPW = _E // _NW
_K = 40
_NCH = _EPW // _K
_SUP = 10
_NSUP = _NCH // _SUP
_NP = 10240
_RPT = _NP // _NS
_BLK = 2048
_EB = 8000


def _sc_edge_pass(T, C, sd_flat, src):
    mesh = plsc.VectorSubcoreMesh(core_axis_name="c", subcore_axis_name="s")

    @functools.partial(
        pl.kernel,
        mesh=mesh,
        out_type=jax.ShapeDtypeStruct((_NC, _NP, _H), jnp.float32),
        scratch_types=[
            pltpu.VMEM((_SUP * 2 * _K,), jnp.int32),
            pltpu.VMEM((_K,), jnp.int32),
            pltpu.VMEM((_K,), jnp.int32),
            pltpu.VMEM((2, 2 * _K, _H), jnp.float32),
            pltpu.VMEM((2, _K, _H), jnp.float32),
            pltpu.VMEM((2, _K, _H), jnp.float32),
            pltpu.VMEM_SHARED((_NP, _H), jnp.float32),
            pltpu.SemaphoreType.DMA,
            pltpu.SemaphoreType.DMA,
            pltpu.SemaphoreType.DMA,
            pltpu.SemaphoreType.DMA,
            pltpu.SemaphoreType.DMA,
            pltpu.SemaphoreType.DMA,
        ],
    )
    def k(t_h, c_h, sd_h, src_h, out_h,
          sdsup, ss0, ss1, bufab, bufc, bufm, acc,
          semg0, semg1, sems0, sems1, semss0, semss1):
        semg = [semg0, semg1]
        sems = [sems0, sems1]
        semss = [semss0, semss1]
        sidxs = [ss0, ss1]
        cid = lax.axis_index("c")
        sid = lax.axis_index("s")
        wid = sid * _NC + cid

        def zrow(r, carry):
            for j in range(_H // _L):
                bufm[0, r, pl.ds(j * _L, _L)] = jnp.zeros((_L,), jnp.float32)
            return carry

        lax.fori_loop(0, _K, zrow, 0)
        for i in range(_RPT // _K):
            pltpu.sync_copy(bufm.at[0], acc.at[pl.ds(sid * _RPT + i * _K, _K)])
        plsc.subcore_barrier()

        sdw = _SUP * 2 * _K
        ebase = wid * _EPW

        def load_super(g):
            pltpu.sync_copy(sd_h.at[pl.ds((wid * _NCH + g * _SUP) * 2 * _K,
                                          sdw)], sdsup)

        def gather_issue(c, r, slot):
            pltpu.async_copy(t_h.at[sdsup.at[pl.ds(r * 2 * _K, 2 * _K)]],
                             bufab.at[slot], semg[slot])
            pltpu.async_copy(c_h.at[pl.ds(ebase + c * _K, _K)],
                             bufc.at[slot], semg[slot])

        def gather_wait(c, r, slot):
            pltpu.make_async_copy(t_h.at[sdsup.at[pl.ds(r * 2 * _K, 2 * _K)]],
                                  bufab.at[slot], semg[slot]).wait()
            pltpu.make_async_copy(c_h.at[pl.ds(ebase + c * _K, _K)],
                                  bufc.at[slot], semg[slot]).wait()

        def step(p, r):
            s = r % 2
            c = p * _SUP + r

            if r < _SUP - 1:
                gather_issue(c + 1, r + 1, 1 - s)

            @pl.when(c >= 2)
            def _():
                pltpu.make_async_copy(
                    bufm.at[s], acc.at[sidxs[s]], sems[s]).wait()

            pltpu.async_copy(src_h.at[pl.ds(ebase + c * _K, _K)],
                             sidxs[s], semss[s])

            gather_wait(c, r, s)

            ba = bufab.at[s]
            bc = bufc.at[s]
            bm = bufm.at[s]

            @plsc.parallel_loop(0, _K, unroll=4)
            def row(e):
                e2 = _K + e
                for j in range(_H // _L):
                    sl = pl.ds(j * _L, _L)
                    v = ba[e, sl] + ba[e2, sl] + bc[e, sl]
                    bm[e, sl] = jnp.maximum(v, 0.0)

            pltpu.make_async_copy(src_h.at[pl.ds(ebase + c * _K, _K)],
                                  sidxs[s], semss[s]).wait()
            pltpu.async_copy(bufm.at[s], acc.at[sidxs[s]], sems[s], add=True)

            if r == _SUP - 1:
                @pl.when(p + 1 < _NSUP)
                def _():
                    load_super(p + 1)
                    gather_issue(c + 1, 0, 1 - s)

        load_super(0)
        gather_issue(0, 0, 0)

        def superchunk(p, carry):
            for r in range(_SUP):
                step(p, r)
            return carry

        lax.fori_loop(0, _NSUP, superchunk, 0)
        pltpu.make_async_copy(bufm.at[0], acc.at[sidxs[0]], sems[0]).wait()
        pltpu.make_async_copy(bufm.at[1], acc.at[sidxs[1]], sems[1]).wait()
        plsc.subcore_barrier()
        pltpu.sync_copy(acc.at[pl.ds(sid * _RPT, _RPT)],
                        out_h.at[cid, pl.ds(sid * _RPT, _RPT)])

    return k(T, C, sd_flat, src)


def _tc_edge(ef, wc0, b0, wc1, b1):
    def body(ef_ref, w0_ref, b0_ref, w1_ref, b1_ref, c0_ref, c1_ref):
        e = ef_ref[...]
        c0_ref[...] = jnp.dot(e, w0_ref[...],
                              preferred_element_type=jnp.float32) + b0_ref[...]
        c1_ref[...] = jnp.dot(e, w1_ref[...],
                              preferred_element_type=jnp.float32) + b1_ref[...]

    wfull = pl.BlockSpec((_ET, _H), lambda i: (0, 0))
    row1 = pl.BlockSpec((1, _H), lambda i: (0, 0))
    eblk = pl.BlockSpec((_EB, _ET), lambda i: (i, 0))
    cblk = pl.BlockSpec((_EB, _H), lambda i: (i, 0))
    return pl.pallas_call(
        body,
        grid=(_E // _EB,),
        in_specs=[eblk, wfull, row1, wfull, row1],
        out_specs=[cblk, cblk],
        out_shape=[
            jax.ShapeDtypeStruct((_E, _H), jnp.float32),
            jax.ShapeDtypeStruct((_E, _H), jnp.float32),
        ],
    )(ef, wc0, b0, wc1, b1)


def _tc_prep(x, w_child, b_child, wa, wb):
    def body(x_ref, wc_ref, bc_ref, wa_ref, wb_ref, t_ref, p_ref):
        i = pl.program_id(0)
        cf = jnp.maximum(
            jnp.dot(x_ref[...], wc_ref[...],
                    preferred_element_type=jnp.float32) + bc_ref[...], 0.0)
        t_ref[0] = jnp.dot(cf, wa_ref[...], preferred_element_type=jnp.float32)
        t_ref[1] = jnp.dot(cf, wb_ref[...], preferred_element_type=jnp.float32)
        s = jnp.sum(cf, axis=0, keepdims=True)

        @pl.when(i == 0)
        def _():
            p_ref[...] = s

        @pl.when(i != 0)
        def _():
            p_ref[...] = p_ref[...] + s

    full = pl.BlockSpec((_H, _H), lambda i: (0, 0))
    row1 = pl.BlockSpec((1, _H), lambda i: (0, 0))
    nblk = pl.BlockSpec((_BLK, _H), lambda i: (i, 0))
    tblk = pl.BlockSpec((2, _BLK, _H), lambda i: (0, i, 0))
    return pl.pallas_call(
        body,
        grid=(_NP // _BLK,),
        in_specs=[nblk, full, row1, full, full],
        out_specs=[tblk, row1],
        out_shape=[
            jax.ShapeDtypeStruct((2, _NP, _H), jnp.float32),
            jax.ShapeDtypeStruct((1, _H), jnp.float32),
        ],
    )(x, w_child, b_child, wa, wb)


def _tc_mid(parts, wa, wb):
    def body(p_ref, wa_ref, wb_ref, t_ref, s_ref):
        i = pl.program_id(0)
        cf = p_ref[0] + p_ref[1]
        t_ref[0] = jnp.dot(cf, wa_ref[...], preferred_element_type=jnp.float32)
        t_ref[1] = jnp.dot(cf, wb_ref[...], preferred_element_type=jnp.float32)
        s = jnp.sum(cf, axis=0, keepdims=True)

        @pl.when(i == 0)
        def _():
            s_ref[...] = s

        @pl.when(i != 0)
        def _():
            s_ref[...] = s_ref[...] + s

    full = pl.BlockSpec((_H, _H), lambda i: (0, 0))
    row1 = pl.BlockSpec((1, _H), lambda i: (0, 0))
    pblk = pl.BlockSpec((2, _BLK, _H), lambda i: (0, i, 0))
    tblk = pl.BlockSpec((2, _BLK, _H), lambda i: (0, i, 0))
    return pl.pallas_call(
        body,
        grid=(_NP // _BLK,),
        in_specs=[pblk, full, full],
        out_specs=[tblk, row1],
        out_shape=[
            jax.ShapeDtypeStruct((2, _NP, _H), jnp.float32),
            jax.ShapeDtypeStruct((1, _H), jnp.float32),
        ],
    )(parts, wa, wb)


def _tc_fin(parts, p0, p1, wp0, wp1, wp2, bp):
    def body(parts_ref, p0_ref, p1_ref, w0_ref, w1_ref, w2_ref, bp_ref,
             out_ref, acc_ref):
        i = pl.program_id(0)
        s = jnp.sum(parts_ref[0] + parts_ref[1], axis=0, keepdims=True)

        @pl.when(i == 0)
        def _():
            acc_ref[...] = s

        @pl.when(i != 0)
        def _():
            acc_ref[...] = acc_ref[...] + s

        @pl.when(i == pl.num_programs(0) - 1)
        def _():
            r = jnp.dot(p0_ref[...], w0_ref[...],
                        preferred_element_type=jnp.float32)
            r = r + jnp.dot(p1_ref[...], w1_ref[...],
                            preferred_element_type=jnp.float32)
            r = r + jnp.dot(acc_ref[...], w2_ref[...],
                            preferred_element_type=jnp.float32)
            out_ref[...] = jnp.maximum(r + bp_ref[...], 0.0)

    full = pl.BlockSpec((_H, _H), lambda i: (0, 0))
    row1 = pl.BlockSpec((1, _H), lambda i: (0, 0))
    pblk = pl.BlockSpec((2, _BLK, _H), lambda i: (0, i, 0))
    return pl.pallas_call(
        body,
        grid=(_NP // _BLK,),
        in_specs=[pblk, row1, row1, full, full, full, row1],
        out_specs=row1,
        out_shape=jax.ShapeDtypeStruct((1, _D), jnp.float32),
        scratch_shapes=[pltpu.VMEM((1, _H), jnp.float32)],
    )(parts, p0, p1, wp0, wp1, wp2, bp)


def kernel(child_feats, child_exists, edge_type_onehot, edge_indices,
           W_child, b_child, W_ne0, b_ne0, W_ne1, b_ne1, W_parent, b_parent):
    x = (child_feats * child_exists)[0]
    x = jnp.concatenate([x, jnp.zeros((_NP - _N, _D), jnp.float32)], axis=0)
    src = edge_indices[0, :, 0]
    dst = edge_indices[0, :, 1]
    sd_flat = jnp.concatenate(
        [src.reshape(_NW, _NCH, _K), dst.reshape(_NW, _NCH, _K) + _NP],
        axis=2).reshape(_E * 2)

    wa0, wb0 = W_ne0[:_H], W_ne0[_H:2 * _H]
    wa1, wb1 = W_ne1[:_H], W_ne1[_H:2 * _H]

    c0, c1 = _tc_edge(edge_type_onehot[0], W_ne0[2 * _H:],
                      b_ne0.reshape(1, _H), W_ne1[2 * _H:],
                      b_ne1.reshape(1, _H))
    c0v = c0
    c1v = c1
    t0, p0 = _tc_prep(x, W_child, b_child.reshape(1, _H), wa0, wb0)
    parts1 = _sc_edge_pass(t0.reshape(2 * _NP, _H), c0v, sd_flat, src)
    t1, p1 = _tc_mid(parts1, wa1, wb1)
    parts2 = _sc_edge_pass(t1.reshape(2 * _NP, _H), c1v, sd_flat, src)
    return _tc_fin(parts2, p0, p1,
                   W_parent[:_H], W_parent[_H:2 * _H], W_parent[2 * _H:],
                   b_parent.reshape(1, _D))


# speedup vs baseline: 3.3458x; 1.0089x over previous
import functools

import jax
import jax.numpy as jnp
from jax import lax
from jax.experimental import pallas as pl
from jax.experimental.pallas import tpu as pltpu
from jax.experimental.pallas import tpu_sc as plsc

_N = 10000
_E = 320000
_D = 128
_H = 128
_ET = 4

_NC = 2
_NS = 16
_L = 16
_NW = _NC * _NS
_EPW = _E // _NW
_K = 40
_NCH = _EPW // _K
_SUP = 10
_NSUP = _NCH // _SUP
_NP = 10240
_RPT = _NP // _NS
_BLK = 2048
_EB = 8000


def _sc_edge_pass(T, C, sd_flat, src):
    mesh = plsc.VectorSubcoreMesh(core_axis_name="c", subcore_axis_name="s")

    @functools.partial(
        pl.kernel,
        mesh=mesh,
        out_type=jax.ShapeDtypeStruct((_NC, _NP, _H), jnp.float32),
        scratch_types=[
            pltpu.VMEM((_SUP * 2 * _K,), jnp.int32),
            pltpu.VMEM((_K,), jnp.int32),
            pltpu.VMEM((_K,), jnp.int32),
            pltpu.VMEM((2, 2 * _K, _H), jnp.float32),
            pltpu.VMEM((2, _K, _H), jnp.float32),
            pltpu.VMEM((2, _K, _H), jnp.float32),
            pltpu.VMEM_SHARED((_NP, _H), jnp.float32),
            pltpu.SemaphoreType.DMA,
            pltpu.SemaphoreType.DMA,
            pltpu.SemaphoreType.DMA,
            pltpu.SemaphoreType.DMA,
            pltpu.SemaphoreType.DMA,
            pltpu.SemaphoreType.DMA,
        ],
    )
    def k(t_h, c_h, sd_h, src_h, out_h,
          sdsup, ss0, ss1, bufab, bufc, bufm, acc,
          semg0, semg1, sems0, sems1, semss0, semss1):
        semg = [semg0, semg1]
        sems = [sems0, sems1]
        semss = [semss0, semss1]
        sidxs = [ss0, ss1]
        cid = lax.axis_index("c")
        sid = lax.axis_index("s")
        wid = sid * _NC + cid

        @plsc.parallel_loop(0, _K, unroll=4)
        def zrow(r):
            for j in range(_H // _L):
                bufm[0, r, pl.ds(j * _L, _L)] = jnp.zeros((_L,), jnp.float32)
        for i in range(_RPT // _K):
            pltpu.sync_copy(bufm.at[0], acc.at[pl.ds(sid * _RPT + i * _K, _K)])
        plsc.subcore_barrier()

        sdw = _SUP * 2 * _K
        ebase = wid * _EPW

        def load_super(g):
            pltpu.sync_copy(sd_h.at[pl.ds((wid * _NCH + g * _SUP) * 2 * _K,
                                          sdw)], sdsup)

        def gather_issue(c, r, slot):
            pltpu.async_copy(t_h.at[sdsup.at[pl.ds(r * 2 * _K, 2 * _K)]],
                             bufab.at[slot], semg[slot])
            pltpu.async_copy(c_h.at[pl.ds(ebase + c * _K, _K)],
                             bufc.at[slot], semg[slot])

        def gather_wait(c, r, slot):
            pltpu.make_async_copy(t_h.at[sdsup.at[pl.ds(r * 2 * _K, 2 * _K)]],
                                  bufab.at[slot], semg[slot]).wait()
            pltpu.make_async_copy(c_h.at[pl.ds(ebase + c * _K, _K)],
                                  bufc.at[slot], semg[slot]).wait()

        def step(p, r):
            s = r % 2
            c = p * _SUP + r

            if r < _SUP - 1:
                gather_issue(c + 1, r + 1, 1 - s)

            @pl.when(c >= 2)
            def _():
                pltpu.make_async_copy(
                    bufm.at[s], acc.at[sidxs[s]], sems[s]).wait()

            pltpu.async_copy(src_h.at[pl.ds(ebase + c * _K, _K)],
                             sidxs[s], semss[s])

            gather_wait(c, r, s)

            ba = bufab.at[s]
            bc = bufc.at[s]
            bm = bufm.at[s]

            @plsc.parallel_loop(0, _K, unroll=4)
            def row(e):
                e2 = _K + e
                for j in range(_H // _L):
                    sl = pl.ds(j * _L, _L)
                    v = ba[e, sl] + ba[e2, sl] + bc[e, sl]
                    bm[e, sl] = jnp.maximum(v, 0.0)

            pltpu.make_async_copy(src_h.at[pl.ds(ebase + c * _K, _K)],
                                  sidxs[s], semss[s]).wait()
            pltpu.async_copy(bufm.at[s], acc.at[sidxs[s]], sems[s], add=True)

            if r == _SUP - 1:
                @pl.when(p + 1 < _NSUP)
                def _():
                    load_super(p + 1)
                    gather_issue(c + 1, 0, 1 - s)

        load_super(0)
        gather_issue(0, 0, 0)

        def superchunk(p, carry):
            for r in range(_SUP):
                step(p, r)
            return carry

        lax.fori_loop(0, _NSUP, superchunk, 0)
        pltpu.make_async_copy(bufm.at[0], acc.at[sidxs[0]], sems[0]).wait()
        pltpu.make_async_copy(bufm.at[1], acc.at[sidxs[1]], sems[1]).wait()
        plsc.subcore_barrier()
        pltpu.sync_copy(acc.at[pl.ds(sid * _RPT, _RPT)],
                        out_h.at[cid, pl.ds(sid * _RPT, _RPT)])

    return k(T, C, sd_flat, src)


def _tc_edge(ef, wc0, b0, wc1, b1):
    def body(ef_ref, w0_ref, b0_ref, w1_ref, b1_ref, c0_ref, c1_ref):
        e = ef_ref[...]
        c0_ref[...] = jnp.dot(e, w0_ref[...],
                              preferred_element_type=jnp.float32) + b0_ref[...]
        c1_ref[...] = jnp.dot(e, w1_ref[...],
                              preferred_element_type=jnp.float32) + b1_ref[...]

    wfull = pl.BlockSpec((_ET, _H), lambda i: (0, 0))
    row1 = pl.BlockSpec((1, _H), lambda i: (0, 0))
    eblk = pl.BlockSpec((_EB, _ET), lambda i: (i, 0))
    cblk = pl.BlockSpec((_EB, _H), lambda i: (i, 0))
    return pl.pallas_call(
        body,
        grid=(_E // _EB,),
        in_specs=[eblk, wfull, row1, wfull, row1],
        out_specs=[cblk, cblk],
        out_shape=[
            jax.ShapeDtypeStruct((_E, _H), jnp.float32),
            jax.ShapeDtypeStruct((_E, _H), jnp.float32),
        ],
    )(ef, wc0, b0, wc1, b1)


def _tc_prep(x, w_child, b_child, wa, wb):
    def body(x_ref, wc_ref, bc_ref, wa_ref, wb_ref, t_ref, p_ref):
        i = pl.program_id(0)
        cf = jnp.maximum(
            jnp.dot(x_ref[...], wc_ref[...],
                    preferred_element_type=jnp.float32) + bc_ref[...], 0.0)
        t_ref[0] = jnp.dot(cf, wa_ref[...], preferred_element_type=jnp.float32)
        t_ref[1] = jnp.dot(cf, wb_ref[...], preferred_element_type=jnp.float32)
        s = jnp.sum(cf, axis=0, keepdims=True)

        @pl.when(i == 0)
        def _():
            p_ref[...] = s

        @pl.when(i != 0)
        def _():
            p_ref[...] = p_ref[...] + s

    full = pl.BlockSpec((_H, _H), lambda i: (0, 0))
    row1 = pl.BlockSpec((1, _H), lambda i: (0, 0))
    nblk = pl.BlockSpec((_BLK, _H), lambda i: (i, 0))
    tblk = pl.BlockSpec((2, _BLK, _H), lambda i: (0, i, 0))
    return pl.pallas_call(
        body,
        grid=(_NP // _BLK,),
        in_specs=[nblk, full, row1, full, full],
        out_specs=[tblk, row1],
        out_shape=[
            jax.ShapeDtypeStruct((2, _NP, _H), jnp.float32),
            jax.ShapeDtypeStruct((1, _H), jnp.float32),
        ],
    )(x, w_child, b_child, wa, wb)


def _tc_mid(parts, wa, wb):
    def body(p_ref, wa_ref, wb_ref, t_ref, s_ref):
        i = pl.program_id(0)
        cf = p_ref[0] + p_ref[1]
        t_ref[0] = jnp.dot(cf, wa_ref[...], preferred_element_type=jnp.float32)
        t_ref[1] = jnp.dot(cf, wb_ref[...], preferred_element_type=jnp.float32)
        s = jnp.sum(cf, axis=0, keepdims=True)

        @pl.when(i == 0)
        def _():
            s_ref[...] = s

        @pl.when(i != 0)
        def _():
            s_ref[...] = s_ref[...] + s

    full = pl.BlockSpec((_H, _H), lambda i: (0, 0))
    row1 = pl.BlockSpec((1, _H), lambda i: (0, 0))
    pblk = pl.BlockSpec((2, _BLK, _H), lambda i: (0, i, 0))
    tblk = pl.BlockSpec((2, _BLK, _H), lambda i: (0, i, 0))
    return pl.pallas_call(
        body,
        grid=(_NP // _BLK,),
        in_specs=[pblk, full, full],
        out_specs=[tblk, row1],
        out_shape=[
            jax.ShapeDtypeStruct((2, _NP, _H), jnp.float32),
            jax.ShapeDtypeStruct((1, _H), jnp.float32),
        ],
    )(parts, wa, wb)


def _tc_fin(parts, p0, p1, wp0, wp1, wp2, bp):
    def body(parts_ref, p0_ref, p1_ref, w0_ref, w1_ref, w2_ref, bp_ref,
             out_ref, acc_ref):
        i = pl.program_id(0)
        s = jnp.sum(parts_ref[0] + parts_ref[1], axis=0, keepdims=True)

        @pl.when(i == 0)
        def _():
            acc_ref[...] = s

        @pl.when(i != 0)
        def _():
            acc_ref[...] = acc_ref[...] + s

        @pl.when(i == pl.num_programs(0) - 1)
        def _():
            r = jnp.dot(p0_ref[...], w0_ref[...],
                        preferred_element_type=jnp.float32)
            r = r + jnp.dot(p1_ref[...], w1_ref[...],
                            preferred_element_type=jnp.float32)
            r = r + jnp.dot(acc_ref[...], w2_ref[...],
                            preferred_element_type=jnp.float32)
            out_ref[...] = jnp.maximum(r + bp_ref[...], 0.0)

    full = pl.BlockSpec((_H, _H), lambda i: (0, 0))
    row1 = pl.BlockSpec((1, _H), lambda i: (0, 0))
    pblk = pl.BlockSpec((2, _BLK, _H), lambda i: (0, i, 0))
    return pl.pallas_call(
        body,
        grid=(_NP // _BLK,),
        in_specs=[pblk, row1, row1, full, full, full, row1],
        out_specs=row1,
        out_shape=jax.ShapeDtypeStruct((1, _D), jnp.float32),
        scratch_shapes=[pltpu.VMEM((1, _H), jnp.float32)],
    )(parts, p0, p1, wp0, wp1, wp2, bp)


def kernel(child_feats, child_exists, edge_type_onehot, edge_indices,
           W_child, b_child, W_ne0, b_ne0, W_ne1, b_ne1, W_parent, b_parent):
    x = (child_feats * child_exists)[0]
    x = jnp.concatenate([x, jnp.zeros((_NP - _N, _D), jnp.float32)], axis=0)
    src = edge_indices[0, :, 0]
    dst = edge_indices[0, :, 1]
    sd_flat = jnp.concatenate(
        [src.reshape(_NW, _NCH, _K), dst.reshape(_NW, _NCH, _K) + _NP],
        axis=2).reshape(_E * 2)

    wa0, wb0 = W_ne0[:_H], W_ne0[_H:2 * _H]
    wa1, wb1 = W_ne1[:_H], W_ne1[_H:2 * _H]

    c0, c1 = _tc_edge(edge_type_onehot[0], W_ne0[2 * _H:],
                      b_ne0.reshape(1, _H), W_ne1[2 * _H:],
                      b_ne1.reshape(1, _H))
    c0v = c0
    c1v = c1
    t0, p0 = _tc_prep(x, W_child, b_child.reshape(1, _H), wa0, wb0)
    parts1 = _sc_edge_pass(t0.reshape(2 * _NP, _H), c0v, sd_flat, src)
    t1, p1 = _tc_mid(parts1, wa1, wb1)
    parts2 = _sc_edge_pass(t1.reshape(2 * _NP, _H), c1v, sd_flat, src)
    return _tc_fin(parts2, p0, p1,
                   W_parent[:_H], W_parent[_H:2 * _H], W_parent[2 * _H:],
                   b_parent.reshape(1, _D))
